# baseline (speedup 1.0000x reference)
import functools

import jax
import jax.numpy as jnp
from jax import lax
from jax.experimental import pallas as pl
from jax.experimental.pallas import tpu as pltpu
from jax.experimental.pallas import tpu_sc as plsc

_N = 10000
_E = 320000
_D = 128
_NC = 2
_NS = 16
_NW = _NC * _NS
_NB = 5
_CHUNK = 64
_CHUNKS = 160
_EPAD = _NW * _CHUNKS * _CHUNK
_NP = 10112
_RPW = 624


def _sc_aggregate(x_tab, pk_p, zeros_n):
    @functools.partial(
        pl.kernel,
        out_type=jax.ShapeDtypeStruct((_NC, _N, _D), jnp.float32),
        mesh=plsc.VectorSubcoreMesh(core_axis_name="c", subcore_axis_name="s"),
        scratch_types=[
            pltpu.VMEM((_NB, 1, _CHUNK), jnp.int32),
            pltpu.VMEM((_NB, _CHUNK), jnp.int32),
            pltpu.VMEM((_NB, _CHUNK), jnp.int32),
            pltpu.VMEM((_CHUNK, _D), jnp.float32),
            pltpu.VMEM((_CHUNK, _D), jnp.float32),
            pltpu.VMEM((_CHUNK, _D), jnp.float32),
            pltpu.VMEM((_CHUNK, _D), jnp.float32),
            pltpu.VMEM((_CHUNK, _D), jnp.float32),
            pltpu.VMEM_SHARED((_NP, _D), jnp.float32),
            pltpu.SemaphoreType.DMA((_NB,)),
            pltpu.SemaphoreType.DMA((_NB,)),
            pltpu.SemaphoreType.DMA((_NB,)),
        ],
    )
    def agg_kernel(x_hbm, pk_hbm, z_hbm, out_hbm,
                   pkr, si_v, di_v, r0, r1, r2, r3, r4, agg_s,
                   psems, gsems, ssems):
        c = lax.axis_index("c")
        s = lax.axis_index("s")
        wid = c * _NS + s
        rows = (r0, r1, r2, r3, r4)
        psem = tuple(psems.at[b] for b in range(_NB))
        gsem = tuple(gsems.at[b] for b in range(_NB))
        ssem = tuple(ssems.at[b] for b in range(_NB))

        def pk_fetch(j, b):
            pltpu.async_copy(pk_hbm.at[wid, j], pkr.at[b], psem[b])

        def pk_wait(j, b):
            pltpu.make_async_copy(pk_hbm.at[wid, j], pkr.at[b],
                                  psem[b]).wait()

        def unpack(b):
            for q in range(_CHUNK // 16):
                v = pkr[b, 0, pl.ds(q * 16, 16)]
                si_v[b, pl.ds(q * 16, 16)] = v & 0xFFFF
                di_v[b, pl.ds(q * 16, 16)] = v >> 16

        def gather(b):
            pltpu.async_copy(x_hbm.at[si_v.at[b]], rows[b], gsem[b])

        def gather_wait(b):
            pltpu.make_async_copy(x_hbm.at[si_v.at[b]], rows[b],
                                  gsem[b]).wait()

        def scatter(b):
            pltpu.async_copy(rows[b], agg_s.at[di_v.at[b]], ssem[b],
                             add=True)

        def scatter_wait(b):
            pltpu.make_async_copy(rows[b], agg_s.at[di_v.at[b]],
                                  ssem[b]).wait()

        for b in range(_NB):
            pk_fetch(b, b)
        for b in range(_NB):
            pk_wait(b, b)
            unpack(b)
            pk_fetch(b + _NB, b)
            gather(b)
        pltpu.sync_copy(z_hbm.at[pl.ds(s * _RPW, _RPW)],
                        agg_s.at[pl.ds(s * _RPW, _RPW)])

        @pl.when(s == _NS - 1)
        def _():
            pltpu.sync_copy(z_hbm.at[pl.ds(_NS * _RPW, _N - _NS * _RPW)],
                            agg_s.at[pl.ds(_NS * _RPW, _N - _NS * _RPW)])

        plsc.subcore_barrier()

        def recycle(j, last, b):
            scatter_wait(b)
            pk_wait(jnp.minimum(j + _NB + b, last), b)
            unpack(b)
            pk_fetch(jnp.minimum(j + 2 * _NB + b, last), b)
            gather(b)

        def body(i, carry):
            j = _NB * i
            last = _CHUNKS - 1
            gather_wait(0)
            scatter(0)
            gather_wait(1)
            scatter(1)
            gather_wait(2)
            scatter(2)
            recycle(j, last, 0)
            gather_wait(3)
            scatter(3)
            recycle(j, last, 1)
            gather_wait(4)
            scatter(4)
            recycle(j, last, 2)
            recycle(j, last, 3)
            recycle(j, last, 4)
            return carry

        lax.fori_loop(0, _CHUNKS // _NB, body, 0)
        for b in range(_NB):
            gather_wait(b)
            pk_wait(_CHUNKS - 1, b)
        plsc.subcore_barrier()
        pltpu.sync_copy(agg_s.at[pl.ds(s * _RPW, _RPW)],
                        out_hbm.at[c, pl.ds(s * _RPW, _RPW)])

        @pl.when(s == _NS - 1)
        def _():
            pltpu.sync_copy(agg_s.at[pl.ds(_NS * _RPW, _N - _NS * _RPW)],
                            out_hbm.at[c, pl.ds(_NS * _RPW, _N - _NS * _RPW)])

    return agg_kernel(x_tab, pk_p, zeros_n)


def _tc_linear(partials, x, w_rel, w_root, b):
    def linear_body(p_ref, x_ref, wr_ref, wt_ref, b_ref, o_ref):
        agg = p_ref[0] + p_ref[1]
        o_ref[...] = (
            jnp.dot(agg, wr_ref[...], preferred_element_type=jnp.float32)
            + jnp.dot(x_ref[...], wt_ref[...], preferred_element_type=jnp.float32)
            + b_ref[...]
        )

    return pl.pallas_call(
        linear_body,
        out_shape=jax.ShapeDtypeStruct((_N, _D), jnp.float32),
    )(partials, x, w_rel, w_root, b.reshape(1, _D))


def kernel(edge_index, features, W1_rel, W1_root, b1, W2_rel, W2_root, b2):
    src = edge_index[0].astype(jnp.int32)
    dst = edge_index[1].astype(jnp.int32)
    pad = _EPAD - _E
    packed = src | (dst << 16)
    ar = jnp.arange(pad, dtype=jnp.int32)
    pad_dump = _N + ar % (_NP - _N)
    pk_p = jnp.concatenate([packed, (ar % _N) | (pad_dump << 16)])
    pk_p = pk_p.reshape(_NW, _CHUNKS, 1, _CHUNK)

    zeros_n = jnp.zeros((_N, _D), jnp.float32)

    p1 = _sc_aggregate(features, pk_p, zeros_n)
    h = _tc_linear(p1, features, W1_rel, W1_root, b1)

    p2 = _sc_aggregate(h, pk_p, zeros_n)
    return _tc_linear(p2, h, W2_rel, W2_root, b2)


# speedup vs baseline: 1.0428x; 1.0428x over previous
import functools

import jax
import jax.numpy as jnp
from jax import lax
from jax.experimental import pallas as pl
from jax.experimental.pallas import tpu as pltpu
from jax.experimental.pallas import tpu_sc as plsc

_N = 10000
_E = 320000
_D = 128
_NC = 2
_NS = 16
_NW = _NC * _NS
_NB = 6
_CHUNK = 48
_CHUNKS = 216
_EPAD = _NW * _CHUNKS * _CHUNK
_NP = 10112
_RPW = 624


def _sc_aggregate(x_tab, pk_p, zeros_n):
    @functools.partial(
        pl.kernel,
        out_type=jax.ShapeDtypeStruct((_NC, _N, _D), jnp.float32),
        mesh=plsc.VectorSubcoreMesh(core_axis_name="c", subcore_axis_name="s"),
        scratch_types=[
            pltpu.VMEM((_NB, 1, _CHUNK), jnp.int32),
            pltpu.VMEM((_NB, _CHUNK), jnp.int32),
            pltpu.VMEM((_NB, _CHUNK), jnp.int32),
            pltpu.VMEM((_CHUNK, _D), jnp.float32),
            pltpu.VMEM((_CHUNK, _D), jnp.float32),
            pltpu.VMEM((_CHUNK, _D), jnp.float32),
            pltpu.VMEM((_CHUNK, _D), jnp.float32),
            pltpu.VMEM((_CHUNK, _D), jnp.float32),
            pltpu.VMEM((_CHUNK, _D), jnp.float32),
            pltpu.VMEM_SHARED((_NP, _D), jnp.float32),
            pltpu.SemaphoreType.DMA((_NB,)),
            pltpu.SemaphoreType.DMA((_NB,)),
            pltpu.SemaphoreType.DMA((_NB,)),
        ],
    )
    def agg_kernel(x_hbm, pk_hbm, z_hbm, out_hbm,
                   pkr, si_v, di_v, r0, r1, r2, r3, r4, r5, agg_s,
                   psems, gsems, ssems):
        c = lax.axis_index("c")
        s = lax.axis_index("s")
        wid = c * _NS + s
        rows = (r0, r1, r2, r3, r4, r5)
        psem = tuple(psems.at[b] for b in range(_NB))
        gsem = tuple(gsems.at[b] for b in range(_NB))
        ssem = tuple(ssems.at[b] for b in range(_NB))

        def pk_fetch(j, b):
            pltpu.async_copy(pk_hbm.at[wid, j], pkr.at[b], psem[b])

        def pk_wait(j, b):
            pltpu.make_async_copy(pk_hbm.at[wid, j], pkr.at[b],
                                  psem[b]).wait()

        def unpack(b):
            for q in range(_CHUNK // 16):
                v = pkr[b, 0, pl.ds(q * 16, 16)]
                si_v[b, pl.ds(q * 16, 16)] = v & 0xFFFF
                di_v[b, pl.ds(q * 16, 16)] = v >> 16

        def gather(b):
            pltpu.async_copy(x_hbm.at[si_v.at[b]], rows[b], gsem[b])

        def gather_wait(b):
            pltpu.make_async_copy(x_hbm.at[si_v.at[b]], rows[b],
                                  gsem[b]).wait()

        def scatter(b):
            pltpu.async_copy(rows[b], agg_s.at[di_v.at[b]], ssem[b],
                             add=True)

        def scatter_wait(b):
            pltpu.make_async_copy(rows[b], agg_s.at[di_v.at[b]],
                                  ssem[b]).wait()

        for b in range(_NB):
            pk_fetch(b, b)
        for b in range(_NB):
            pk_wait(b, b)
            unpack(b)
            pk_fetch(b + _NB, b)
            gather(b)
        pltpu.sync_copy(z_hbm.at[pl.ds(s * _RPW, _RPW)],
                        agg_s.at[pl.ds(s * _RPW, _RPW)])

        @pl.when(s == _NS - 1)
        def _():
            pltpu.sync_copy(z_hbm.at[pl.ds(_NS * _RPW, _N - _NS * _RPW)],
                            agg_s.at[pl.ds(_NS * _RPW, _N - _NS * _RPW)])

        plsc.subcore_barrier()

        def recycle(j, last, b):
            scatter_wait(b)
            pk_wait(jnp.minimum(j + _NB + b, last), b)
            unpack(b)
            pk_fetch(jnp.minimum(j + 2 * _NB + b, last), b)
            gather(b)

        def body(i, carry):
            j = _NB * i
            last = _CHUNKS - 1
            gather_wait(0)
            scatter(0)
            gather_wait(1)
            scatter(1)
            gather_wait(2)
            scatter(2)
            recycle(j, last, 0)
            gather_wait(3)
            scatter(3)
            recycle(j, last, 1)
            gather_wait(4)
            scatter(4)
            recycle(j, last, 2)
            gather_wait(5)
            scatter(5)
            recycle(j, last, 3)
            recycle(j, last, 4)
            recycle(j, last, 5)
            return carry

        lax.fori_loop(0, _CHUNKS // _NB, body, 0)
        for b in range(_NB):
            gather_wait(b)
            pk_wait(_CHUNKS - 1, b)
        plsc.subcore_barrier()
        pltpu.sync_copy(agg_s.at[pl.ds(s * _RPW, _RPW)],
                        out_hbm.at[c, pl.ds(s * _RPW, _RPW)])

        @pl.when(s == _NS - 1)
        def _():
            pltpu.sync_copy(agg_s.at[pl.ds(_NS * _RPW, _N - _NS * _RPW)],
                            out_hbm.at[c, pl.ds(_NS * _RPW, _N - _NS * _RPW)])

    return agg_kernel(x_tab, pk_p, zeros_n)


def _tc_linear(partials, x, w_rel, w_root, b):
    def linear_body(p_ref, x_ref, wr_ref, wt_ref, b_ref, o_ref):
        agg = p_ref[0] + p_ref[1]
        o_ref[...] = (
            jnp.dot(agg, wr_ref[...], preferred_element_type=jnp.float32)
            + jnp.dot(x_ref[...], wt_ref[...], preferred_element_type=jnp.float32)
            + b_ref[...]
        )

    return pl.pallas_call(
        linear_body,
        out_shape=jax.ShapeDtypeStruct((_N, _D), jnp.float32),
    )(partials, x, w_rel, w_root, b.reshape(1, _D))


def kernel(edge_index, features, W1_rel, W1_root, b1, W2_rel, W2_root, b2):
    src = edge_index[0].astype(jnp.int32)
    dst = edge_index[1].astype(jnp.int32)
    pad = _EPAD - _E
    packed = src | (dst << 16)
    ar = jnp.arange(pad, dtype=jnp.int32)
    pad_dump = _N + ar % (_NP - _N)
    pk_p = jnp.concatenate([packed, (ar % _N) | (pad_dump << 16)])
    pk_p = pk_p.reshape(_NW, _CHUNKS, 1, _CHUNK)

    zeros_n = jnp.zeros((_N, _D), jnp.float32)

    p1 = _sc_aggregate(features, pk_p, zeros_n)
    h = _tc_linear(p1, features, W1_rel, W1_root, b1)

    p2 = _sc_aggregate(h, pk_p, zeros_n)
    return _tc_linear(p2, h, W2_rel, W2_root, b2)
